# baseline (speedup 1.0000x reference)
import functools

import jax
import jax.numpy as jnp
from jax import lax
from jax.experimental import pallas as pl
from jax.experimental.pallas import tpu as pltpu
from jax.experimental.pallas import tpu_sc as plsc

_F = 26
_D = 64
_B = 4096
_N = _B * _F

_NC, _NS = 2, 16
_NW = _NC * _NS
_PER_W = _N // _NW
_CHUNK = 128
_NCHUNK = _PER_W // _CHUNK
_NBUF = 4

_mesh = plsc.VectorSubcoreMesh(core_axis_name="c", subcore_axis_name="s")


@functools.partial(
    pl.kernel,
    mesh=_mesh,
    out_type=(
        jax.ShapeDtypeStruct((_N, _D), jnp.float32),
        jax.ShapeDtypeStruct((_N, _D), jnp.float32),
    ),
    scratch_types=[
        pltpu.VMEM((_NCHUNK, _CHUNK), jnp.int32),
        pltpu.VMEM((_NBUF, _CHUNK, _D), jnp.float32),
    ]
    + [pltpu.SemaphoreType.DMA] * (2 * _NBUF),
    compiler_params=pltpu.CompilerParams(use_tc_tiling_on_sc=False),
)
def _gather(idx_hbm, table_hbm, out0_hbm, out1_hbm, idx_v, rows_v, *sems):
    gsems, osems = sems[:_NBUF], sems[_NBUF:]
    wid = lax.axis_index("s") * _NC + lax.axis_index("c")
    base = wid * _PER_W
    pltpu.sync_copy(idx_hbm.at[wid], idx_v)
    pend_g = [None] * _NBUF
    pend_o = [None] * _NBUF

    def _writeback(j):
        b = j % _NBUF
        pend_g[b].wait()
        dst = pl.ds(base + j * _CHUNK, _CHUNK)
        pend_o[b] = (
            pltpu.async_copy(rows_v.at[b], out0_hbm.at[dst], osems[b]),
            pltpu.async_copy(rows_v.at[b], out1_hbm.at[dst], osems[b]),
        )

    for j in range(_NCHUNK):
        b = j % _NBUF
        if pend_o[b] is not None:
            pend_o[b][0].wait()
            pend_o[b][1].wait()
        pend_g[b] = pltpu.async_copy(
            table_hbm.at[idx_v.at[j]], rows_v.at[b], gsems[b])
        if j >= _NBUF - 1:
            _writeback(j - (_NBUF - 1))
    for j in range(max(0, _NCHUNK - (_NBUF - 1)), _NCHUNK):
        _writeback(j)
    for b in range(_NBUF):
        if pend_o[b] is not None:
            pend_o[b][0].wait()
            pend_o[b][1].wait()


def kernel(X, table):
    idx = X.reshape(_NW, _NCHUNK, _CHUNK)
    flat0, flat1 = _gather(idx, table)
    return (flat0.reshape(_B, _F * _D), flat1.reshape(_B, _F * _D))


# speedup vs baseline: 1.4789x; 1.4789x over previous
import functools

import jax
import jax.numpy as jnp
from jax import lax
from jax.experimental import pallas as pl
from jax.experimental.pallas import tpu as pltpu
from jax.experimental.pallas import tpu_sc as plsc

_F = 26
_D = 64
_B = 4096
_N = _B * _F

_NC, _NS = 2, 16
_NW = _NC * _NS
_PER_W = _N // _NW
_CHUNK = 128
_NCHUNK = _PER_W // _CHUNK
_NBUF = 4

_mesh = plsc.VectorSubcoreMesh(core_axis_name="c", subcore_axis_name="s")


@functools.partial(
    pl.kernel,
    mesh=_mesh,
    out_type=(
        jax.ShapeDtypeStruct((_N, _D), jnp.float32),
        jax.ShapeDtypeStruct((_N, _D), jnp.float32),
    ),
    scratch_types=[
        pltpu.VMEM((_NCHUNK, _CHUNK), jnp.int32),
        pltpu.VMEM((_NBUF, _CHUNK, _D), jnp.float32),
    ]
    + [pltpu.SemaphoreType.DMA] * (2 * _NBUF),
    compiler_params=pltpu.CompilerParams(use_tc_tiling_on_sc=False),
)
def _gather(idx_hbm, table_hbm, out0_hbm, out1_hbm, idx_v, rows_v, *sems):
    gsems, osems = sems[:_NBUF], sems[_NBUF:]
    wid = lax.axis_index("s") * _NC + lax.axis_index("c")
    base = wid * _PER_W
    pltpu.sync_copy(idx_hbm.at[wid], idx_v)
    pend_g = [None] * _NBUF
    pend_o = [None] * _NBUF

    def _writeback(j):
        b = j % _NBUF
        pend_g[b].wait()
        dst = pl.ds(base + j * _CHUNK, _CHUNK)
        pend_o[b] = (
            pltpu.async_copy(rows_v.at[b], out0_hbm.at[dst], osems[b]),
            pltpu.async_copy(rows_v.at[b], out1_hbm.at[dst], osems[b]),
        )

    for j in range(_NCHUNK):
        b = j % _NBUF
        if pend_o[b] is not None:
            pend_o[b][0].wait()
            pend_o[b][1].wait()
        pend_g[b] = pltpu.async_copy(
            table_hbm.at[idx_v.at[j]], rows_v.at[b], gsems[b])
        if j >= _NBUF - 1:
            _writeback(j - (_NBUF - 1))
    for j in range(max(0, _NCHUNK - (_NBUF - 1)), _NCHUNK):
        _writeback(j)
    for b in range(_NBUF):
        if pend_o[b] is not None:
            pend_o[b][0].wait()
            pend_o[b][1].wait()


def _depermute(flat):
    return (
        flat.reshape(_B // 8, (_F * _D) // 128, 8, 128)
        .transpose(0, 2, 1, 3)
        .reshape(_B, _F * _D)
    )


def kernel(X, table):
    idx = (
        X.reshape(_NW, 16, 8, (_F * _D) // 128, 2)
        .transpose(0, 1, 3, 2, 4)
        .reshape(_NW, _NCHUNK, _CHUNK)
    )
    flat0, flat1 = _gather(idx, table)
    return (_depermute(flat0), _depermute(flat1))


# speedup vs baseline: 1.5163x; 1.0253x over previous
import functools

import jax
import jax.numpy as jnp
from jax import lax
from jax.experimental import pallas as pl
from jax.experimental.pallas import tpu as pltpu
from jax.experimental.pallas import tpu_sc as plsc

_F = 26
_D = 64
_B = 4096
_N = _B * _F

_NC, _NS = 2, 16
_NW = _NC * _NS
_PER_W = _N // _NW
_SPW = _B // _NW
_CHUNK = 128
_NCHUNK = _PER_W // _CHUNK
_NBUF = 4
_NSLAB = _SPW // 8
_NTC = (_F * _D) // 128

_mesh = plsc.VectorSubcoreMesh(core_axis_name="c", subcore_axis_name="s")


@functools.partial(
    pl.kernel,
    mesh=_mesh,
    out_type=(
        jax.ShapeDtypeStruct((_N, _D), jnp.float32),
        jax.ShapeDtypeStruct((_N, _D), jnp.float32),
    ),
    scratch_types=[
        pltpu.VMEM((_F, _SPW), jnp.int32),
        pltpu.VMEM((_NCHUNK, _CHUNK), jnp.int32),
        pltpu.VMEM((_NBUF, _CHUNK, _D), jnp.float32),
    ]
    + [pltpu.SemaphoreType.DMA] * (2 * _NBUF),
    compiler_params=pltpu.CompilerParams(
        use_tc_tiling_on_sc=False, needs_layout_passes=False),
)
def _gather(xt_hbm, table_hbm, out0_hbm, out1_hbm, idx_v, idxp_v, rows_v,
            *sems):
    gsems, osems = sems[:_NBUF], sems[_NBUF:]
    wid = lax.axis_index("s") * _NC + lax.axis_index("c")
    base = wid * _PER_W
    pltpu.sync_copy(xt_hbm.at[:, pl.ds(wid * _SPW, _SPW)], idx_v)

    lane = lax.broadcasted_iota(jnp.int32, (16,), 0)
    pat_e = lane & 1
    pat_r = lane >> 1

    def _slab(j, _):
        for g in range(_NTC):
            vals = plsc.load_gather(idx_v, [pat_e + 2 * g, pat_r + j * 8])
            q16 = j * _NTC + g
            idxp_v[q16 // 8, pl.ds((q16 % 8) * 16, 16)] = vals
        return _

    lax.fori_loop(0, _NSLAB, _slab, 0)

    pend_g = [None] * _NBUF
    pend_o = [None] * _NBUF

    def _writeback(j):
        b = j % _NBUF
        pend_g[b].wait()
        dst = pl.ds(base + j * _CHUNK, _CHUNK)
        pend_o[b] = (
            pltpu.async_copy(rows_v.at[b], out0_hbm.at[dst], osems[b]),
            pltpu.async_copy(rows_v.at[b], out1_hbm.at[dst], osems[b]),
        )

    for j in range(_NCHUNK):
        b = j % _NBUF
        if pend_o[b] is not None:
            pend_o[b][0].wait()
            pend_o[b][1].wait()
        pend_g[b] = pltpu.async_copy(
            table_hbm.at[idxp_v.at[j]], rows_v.at[b], gsems[b])
        if j >= _NBUF - 1:
            _writeback(j - (_NBUF - 1))
    for j in range(max(0, _NCHUNK - (_NBUF - 1)), _NCHUNK):
        _writeback(j)
    for b in range(_NBUF):
        if pend_o[b] is not None:
            pend_o[b][0].wait()
            pend_o[b][1].wait()


def _depermute(flat):
    return (
        flat.reshape(_B // 8, _NTC, 8, 128)
        .transpose(0, 2, 1, 3)
        .reshape(_B, _F * _D)
    )


def kernel(X, table):
    flat0, flat1 = _gather(X.T, table)
    return (_depermute(flat0), _depermute(flat1))


# speedup vs baseline: 1.5363x; 1.0132x over previous
import functools

import jax
import jax.numpy as jnp
from jax import lax
from jax.experimental import pallas as pl
from jax.experimental.pallas import tpu as pltpu
from jax.experimental.pallas import tpu_sc as plsc

_F = 26
_D = 64
_B = 4096
_N = _B * _F

_NC, _NS = 2, 16
_NW = _NC * _NS
_PER_W = _N // _NW
_SPW = _B // _NW
_CHUNK = 256
_NCHUNK = _PER_W // _CHUNK
_NBUF = 4
_NSLAB = _SPW // 8
_NTC = (_F * _D) // 128

_mesh = plsc.VectorSubcoreMesh(core_axis_name="c", subcore_axis_name="s")


@functools.partial(
    pl.kernel,
    mesh=_mesh,
    out_type=(
        jax.ShapeDtypeStruct((_N, _D), jnp.float32),
        jax.ShapeDtypeStruct((_N, _D), jnp.float32),
    ),
    scratch_types=[
        pltpu.VMEM((_F, _SPW), jnp.int32),
        pltpu.VMEM((_NCHUNK, _CHUNK), jnp.int32),
        pltpu.VMEM((_NBUF, _CHUNK, _D), jnp.float32),
    ]
    + [pltpu.SemaphoreType.DMA] * (2 * _NBUF),
    compiler_params=pltpu.CompilerParams(
        use_tc_tiling_on_sc=False, needs_layout_passes=False),
)
def _gather(xt_hbm, table_hbm, out0_hbm, out1_hbm, idx_v, idxp_v, rows_v,
            *sems):
    gsems, osems = sems[:_NBUF], sems[_NBUF:]
    wid = lax.axis_index("s") * _NC + lax.axis_index("c")
    base = wid * _PER_W
    pltpu.sync_copy(xt_hbm.at[:, pl.ds(wid * _SPW, _SPW)], idx_v)

    lane = lax.broadcasted_iota(jnp.int32, (16,), 0)
    pat_e = lane & 1
    pat_r = lane >> 1

    def _slab(j, _):
        for g in range(_NTC):
            vals = plsc.load_gather(idx_v, [pat_e + 2 * g, pat_r + j * 8])
            q16 = j * _NTC + g
            gpr = _CHUNK // 16
            idxp_v[q16 // gpr, pl.ds((q16 % gpr) * 16, 16)] = vals
        return _

    lax.fori_loop(0, _NSLAB, _slab, 0)

    pend_g = [None] * _NBUF
    pend_o = [None] * _NBUF

    def _writeback(j):
        b = j % _NBUF
        pend_g[b].wait()
        dst = pl.ds(base + j * _CHUNK, _CHUNK)
        pend_o[b] = (
            pltpu.async_copy(rows_v.at[b], out0_hbm.at[dst], osems[b]),
            pltpu.async_copy(rows_v.at[b], out1_hbm.at[dst], osems[b]),
        )

    for j in range(_NCHUNK):
        b = j % _NBUF
        if pend_o[b] is not None:
            pend_o[b][0].wait()
            pend_o[b][1].wait()
        pend_g[b] = pltpu.async_copy(
            table_hbm.at[idxp_v.at[j]], rows_v.at[b], gsems[b])
        if j >= _NBUF - 1:
            _writeback(j - (_NBUF - 1))
    for j in range(max(0, _NCHUNK - (_NBUF - 1)), _NCHUNK):
        _writeback(j)
    for b in range(_NBUF):
        if pend_o[b] is not None:
            pend_o[b][0].wait()
            pend_o[b][1].wait()


def _depermute(flat):
    return (
        flat.reshape(_B // 8, _NTC, 8, 128)
        .transpose(0, 2, 1, 3)
        .reshape(_B, _F * _D)
    )


def kernel(X, table):
    flat0, flat1 = _gather(X.T, table)
    return (_depermute(flat0), _depermute(flat1))


# speedup vs baseline: 1.5506x; 1.0093x over previous
import functools

import jax
import jax.numpy as jnp
from jax import lax
from jax.experimental import pallas as pl
from jax.experimental.pallas import tpu as pltpu
from jax.experimental.pallas import tpu_sc as plsc

_F = 26
_D = 64
_B = 4096
_N = _B * _F

_NC, _NS = 2, 16
_NW = _NC * _NS
_PER_W = _N // _NW
_SPW = _B // _NW
_CHUNK = 416
_NCHUNK = _PER_W // _CHUNK
_NBUF = 4
_NSLAB = _SPW // 8
_NTC = (_F * _D) // 128

_mesh = plsc.VectorSubcoreMesh(core_axis_name="c", subcore_axis_name="s")


@functools.partial(
    pl.kernel,
    mesh=_mesh,
    out_type=(
        jax.ShapeDtypeStruct((_N, _D), jnp.float32),
        jax.ShapeDtypeStruct((_N, _D), jnp.float32),
    ),
    scratch_types=[
        pltpu.VMEM((_F, _SPW), jnp.int32),
        pltpu.VMEM((_NCHUNK, _CHUNK), jnp.int32),
        pltpu.VMEM((_NBUF, _CHUNK, _D), jnp.float32),
    ]
    + [pltpu.SemaphoreType.DMA] * (2 * _NBUF),
    compiler_params=pltpu.CompilerParams(
        use_tc_tiling_on_sc=False, needs_layout_passes=False),
)
def _gather(xt_hbm, table_hbm, out0_hbm, out1_hbm, idx_v, idxp_v, rows_v,
            *sems):
    gsems, osems = sems[:_NBUF], sems[_NBUF:]
    wid = lax.axis_index("s") * _NC + lax.axis_index("c")
    base = wid * _PER_W
    pltpu.sync_copy(xt_hbm.at[:, pl.ds(wid * _SPW, _SPW)], idx_v)

    lane = lax.broadcasted_iota(jnp.int32, (16,), 0)
    pat_e = lane & 1
    pat_r = lane >> 1

    def _slab(j, _):
        for g in range(_NTC):
            vals = plsc.load_gather(idx_v, [pat_e + 2 * g, pat_r + j * 8])
            q16 = j * _NTC + g
            gpr = _CHUNK // 16
            idxp_v[q16 // gpr, pl.ds((q16 % gpr) * 16, 16)] = vals
        return _

    lax.fori_loop(0, _NSLAB, _slab, 0)

    pend_g = [None] * _NBUF
    pend_o = [None] * _NBUF

    def _writeback(j):
        b = j % _NBUF
        pend_g[b].wait()
        dst = pl.ds(base + j * _CHUNK, _CHUNK)
        pend_o[b] = (
            pltpu.async_copy(rows_v.at[b], out0_hbm.at[dst], osems[b]),
            pltpu.async_copy(rows_v.at[b], out1_hbm.at[dst], osems[b]),
        )

    for j in range(_NCHUNK):
        b = j % _NBUF
        if pend_o[b] is not None:
            pend_o[b][0].wait()
            pend_o[b][1].wait()
        pend_g[b] = pltpu.async_copy(
            table_hbm.at[idxp_v.at[j]], rows_v.at[b], gsems[b])
        if j >= _NBUF - 1:
            _writeback(j - (_NBUF - 1))
    for j in range(max(0, _NCHUNK - (_NBUF - 1)), _NCHUNK):
        _writeback(j)
    for b in range(_NBUF):
        if pend_o[b] is not None:
            pend_o[b][0].wait()
            pend_o[b][1].wait()


def _depermute(flat):
    return (
        flat.reshape(_B // 8, _NTC, 8, 128)
        .transpose(0, 2, 1, 3)
        .reshape(_B, _F * _D)
    )


def kernel(X, table):
    flat0, flat1 = _gather(X.T, table)
    return (_depermute(flat0), _depermute(flat1))
